# baseline (speedup 1.0000x reference)
import functools

import jax
import jax.numpy as jnp
from jax import lax
from jax.experimental import pallas as pl
from jax.experimental.pallas import tpu as pltpu
from jax.experimental.pallas import tpu_sc as plsc

NUM_EMBEDDINGS = 1000000
EMBEDDING_DIM = 64
B0 = 16384
B1 = 20
BATCH = B0 * B1
PAIR_W = 2 * EMBEDDING_DIM

_INFO = plsc.get_sparse_core_info()
NC = _INFO.num_cores
NS = _INFO.num_subcores
NW = NC * NS
BPW = BATCH // NW

GW = 128
NGROUPS = BPW // GW
SLAB = 640
GPS = SLAB // GW
NSLABS = BPW // SLAB
SLAB_ROWS = SLAB // 2
IDX_ROWS = BPW // 128
L = 16

_mesh = plsc.VectorSubcoreMesh(core_axis_name="c", subcore_axis_name="s")


@functools.partial(
    pl.kernel,
    mesh=_mesh,
    compiler_params=pltpu.CompilerParams(
        use_tc_tiling_on_sc=False, needs_layout_passes=False
    ),
    out_type=jax.ShapeDtypeStruct((BATCH // 2, PAIR_W), jnp.float32),
    scratch_types=[
        pltpu.VMEM((IDX_ROWS, 128), jnp.int32),
        pltpu.VMEM((IDX_ROWS, 128), jnp.int32),
        pltpu.VMEM((2, GW, PAIR_W), jnp.float32),
        pltpu.VMEM((SLAB_ROWS, PAIR_W), jnp.float32),
        pltpu.SemaphoreType.DMA,
        pltpu.SemaphoreType.DMA,
    ],
)
def _gather_kernel(idx_hbm, pidx_hbm, wpair_hbm, out_hbm, idx_v, pidx_v,
                   pairs_v, sel_v, gsem, osem):
    wid = lax.axis_index("s") * NC + lax.axis_index("c")
    idx_row_base = pl.multiple_of(wid * IDX_ROWS, 8)
    out_row_base = wid * (BPW // 2)

    pltpu.sync_copy(idx_hbm.at[pl.ds(idx_row_base, IDX_ROWS)], idx_v)
    pltpu.sync_copy(pidx_hbm.at[pl.ds(idx_row_base, IDX_ROWS)], pidx_v)

    def fire_gather(g, b):
        pltpu.async_copy(wpair_hbm.at[pidx_v.at[g]], pairs_v.at[b], gsem)

    def wait_gather(g, b):
        pltpu.make_async_copy(
            wpair_hbm.at[pidx_v.at[g]], pairs_v.at[b], gsem
        ).wait()

    def out_slice(s):
        o = pl.multiple_of(out_row_base + s * SLAB_ROWS, 8)
        return out_hbm.at[pl.ds(o, SLAB_ROWS)]

    def start_out(s):
        pltpu.async_copy(sel_v, out_slice(s), osem)

    def wait_out(s):
        pltpu.make_async_copy(sel_v, out_slice(s), osem).wait()

    lanes = jax.lax.broadcasted_iota(jnp.int32, (L,), 0)

    def select_group(g, jj, b):
        for k in range(GW // L):
            lookup = (jj * GW + k * L) + lanes
            srow = lookup >> 1
            scol0 = (lookup & 1) << 6
            rowv = (k * L) + lanes
            hv = idx_v[g, pl.ds(k * L, L)]
            hoff0 = (hv & 1) << 6

            def col_body(ci, carry):
                colv, scolv = carry
                for _ in range(4):
                    v = plsc.load_gather(pairs_v.at[b], [rowv, colv])
                    plsc.store_scatter(sel_v, [srow, scolv], v)
                    colv = colv + 1
                    scolv = scolv + 1
                return (colv, scolv)

            lax.fori_loop(0, EMBEDDING_DIM // 4, col_body, (hoff0, scol0))

    def slab(s, par0):
        @pl.when(s > 0)
        def _():
            wait_out(s - 1)

        for jj in range(GPS):
            g = s * GPS + jj
            b = (par0 + jj) & 1
            wait_gather(g, b)

            @pl.when(g + 1 < NGROUPS)
            def _():
                fire_gather(g + 1, 1 - b)

            select_group(g, jj, b)
        start_out(s)

    fire_gather(0, 0)

    def body(t, carry):
        slab(2 * t, 0)
        slab(2 * t + 1, 1)
        return carry

    lax.fori_loop(0, NSLABS // 2, body, 0)

    wait_out(NSLABS - 1)


def kernel(input_, weight):
    idx = input_.reshape(BATCH // 128, 128)
    pidx = (idx >> 1)
    wpair = weight.reshape(NUM_EMBEDDINGS // 2, PAIR_W)
    out = _gather_kernel(idx, pidx, wpair)
    return out.reshape(B0, B1, EMBEDDING_DIM)


# speedup vs baseline: 1.7433x; 1.7433x over previous
import functools

import jax
import jax.numpy as jnp
from jax import lax
from jax.experimental import pallas as pl
from jax.experimental.pallas import tpu as pltpu
from jax.experimental.pallas import tpu_sc as plsc

NUM_EMBEDDINGS = 1000000
EMBEDDING_DIM = 64
B0 = 16384
B1 = 20
BATCH = B0 * B1

_INFO = plsc.get_sparse_core_info()
NC = _INFO.num_cores
NS = _INFO.num_subcores
NW = NC * NS
BPW = BATCH // NW

GW = 128
HG = GW // 2
NGROUPS = BPW // GW
IDX_ROWS = BPW // 128
L = 16

_mesh = plsc.VectorSubcoreMesh(core_axis_name="c", subcore_axis_name="s")


@functools.partial(
    pl.kernel,
    mesh=_mesh,
    compiler_params=pltpu.CompilerParams(
        use_tc_tiling_on_sc=False, needs_layout_passes=False
    ),
    out_type=jax.ShapeDtypeStruct((BATCH // 2, 2 * EMBEDDING_DIM), jnp.float32),
    scratch_types=[
        pltpu.VMEM((IDX_ROWS, 128), jnp.int32),
        pltpu.VMEM((2, 2, HG), jnp.int32),
        pltpu.VMEM((2, 2, HG, EMBEDDING_DIM), jnp.float32),
        pltpu.SemaphoreType.DMA,
        pltpu.SemaphoreType.DMA,
    ],
)
def _gather_kernel(idx_hbm, w64_hbm, out_hbm, idx_v, lists_v, rows_v,
                   gsem, osem):
    wid = lax.axis_index("s") * NC + lax.axis_index("c")
    idx_row_base = pl.multiple_of(wid * IDX_ROWS, 8)
    out_row_base = wid * (BPW // 2)

    pltpu.sync_copy(idx_hbm.at[pl.ds(idx_row_base, IDX_ROWS)], idx_v)

    lanes = jax.lax.broadcasted_iota(jnp.int32, (L,), 0)

    def build_lists(g, b):
        for k in range(HG // L):
            src = 2 * k * L + 2 * lanes
            ev = plsc.load_gather(idx_v.at[g], [src])
            od = plsc.load_gather(idx_v.at[g], [src + 1])
            plsc.store_scatter(lists_v.at[b].at[0], [k * L + lanes], ev)
            plsc.store_scatter(lists_v.at[b].at[1], [k * L + lanes], od)

    def fire_gathers(b):
        for p in range(2):
            pltpu.async_copy(
                w64_hbm.at[lists_v.at[b].at[p]], rows_v.at[b].at[p], gsem
            )

    def wait_gathers(b):
        for p in range(2):
            pltpu.make_async_copy(
                w64_hbm.at[lists_v.at[b].at[p]], rows_v.at[b].at[p], gsem
            ).wait()

    def out_windows(g):
        o = pl.multiple_of(out_row_base + g * HG, 8)
        win = out_hbm.at[pl.ds(o, HG)]
        return (
            win.at[:, pl.ds(0, EMBEDDING_DIM)],
            win.at[:, pl.ds(EMBEDDING_DIM, EMBEDDING_DIM)],
        )

    def start_out(g, b):
        we, wo = out_windows(g)
        pltpu.async_copy(rows_v.at[b].at[0], we, osem)
        pltpu.async_copy(rows_v.at[b].at[1], wo, osem)

    def wait_out(g, b):
        we, wo = out_windows(g)
        pltpu.make_async_copy(rows_v.at[b].at[0], we, osem).wait()
        pltpu.make_async_copy(rows_v.at[b].at[1], wo, osem).wait()

    build_lists(0, 0)
    fire_gathers(0)

    def proc(g, b, first):
        @pl.when(g + 1 < NGROUPS)
        def _():
            build_lists(g + 1, 1 - b)

        wait_gathers(b)
        if not first:
            wait_out(g - 1, 1 - b)

        @pl.when(g + 1 < NGROUPS)
        def _():
            fire_gathers(1 - b)

        start_out(g, b)

    proc(0, 0, True)
    proc(1, 1, False)

    def body(t, carry):
        proc(2 * t, 0, False)
        proc(2 * t + 1, 1, False)
        return carry

    lax.fori_loop(1, NGROUPS // 2, body, 0)

    wait_out(NGROUPS - 1, 1)


def kernel(input_, weight):
    idx = input_.reshape(BATCH // 128, 128)
    out = _gather_kernel(idx, weight)
    return out.reshape(B0, B1, EMBEDDING_DIM)


# speedup vs baseline: 1.8103x; 1.0384x over previous
import functools

import jax
import jax.numpy as jnp
from jax import lax
from jax.experimental import pallas as pl
from jax.experimental.pallas import tpu as pltpu
from jax.experimental.pallas import tpu_sc as plsc

NUM_EMBEDDINGS = 1000000
EMBEDDING_DIM = 64
B0 = 16384
B1 = 20
BATCH = B0 * B1

_INFO = plsc.get_sparse_core_info()
NC = _INFO.num_cores
NS = _INFO.num_subcores
NW = NC * NS
BPW = BATCH // NW

GW = 128
HG = GW // 2
NGROUPS = BPW // GW
IDX_ROWS = BPW // 128
L = 16

_mesh = plsc.VectorSubcoreMesh(core_axis_name="c", subcore_axis_name="s")


@functools.partial(
    pl.kernel,
    mesh=_mesh,
    compiler_params=pltpu.CompilerParams(
        use_tc_tiling_on_sc=False, needs_layout_passes=False
    ),
    out_type=jax.ShapeDtypeStruct((BATCH // 2, 2 * EMBEDDING_DIM), jnp.float32),
    scratch_types=[
        pltpu.VMEM((IDX_ROWS, 128), jnp.int32),
        pltpu.VMEM((NGROUPS, 2, HG), jnp.int32),
        pltpu.VMEM((4, 2, HG, EMBEDDING_DIM), jnp.float32),
        pltpu.SemaphoreType.DMA,
        pltpu.SemaphoreType.DMA,
    ],
)
def _gather_kernel(idx_hbm, w64_hbm, out_hbm, idx_v, lists_v, rows_v,
                   gsem, osem):
    wid = lax.axis_index("s") * NC + lax.axis_index("c")
    idx_row_base = pl.multiple_of(wid * IDX_ROWS, 8)
    out_row_base = wid * (BPW // 2)

    pltpu.sync_copy(idx_hbm.at[pl.ds(idx_row_base, IDX_ROWS)], idx_v)

    lanes = jax.lax.broadcasted_iota(jnp.int32, (L,), 0)

    def build_lists(g):
        for k in range(HG // L):
            src = 2 * k * L + 2 * lanes
            ev = plsc.load_gather(idx_v.at[g], [src])
            od = plsc.load_gather(idx_v.at[g], [src + 1])
            plsc.store_scatter(lists_v.at[g].at[0], [k * L + lanes], ev)
            plsc.store_scatter(lists_v.at[g].at[1], [k * L + lanes], od)

    def fire_gathers(g, b):
        for p in range(2):
            pltpu.async_copy(
                w64_hbm.at[lists_v.at[g].at[p]], rows_v.at[b].at[p], gsem
            )

    def wait_gathers(g, b):
        for p in range(2):
            pltpu.make_async_copy(
                w64_hbm.at[lists_v.at[g].at[p]], rows_v.at[b].at[p], gsem
            ).wait()

    def out_windows(g):
        o = pl.multiple_of(out_row_base + g * HG, 8)
        win = out_hbm.at[pl.ds(o, HG)]
        return (
            win.at[:, pl.ds(0, EMBEDDING_DIM)],
            win.at[:, pl.ds(EMBEDDING_DIM, EMBEDDING_DIM)],
        )

    def start_out(g, b):
        we, wo = out_windows(g)
        pltpu.async_copy(rows_v.at[b].at[0], we, osem)
        pltpu.async_copy(rows_v.at[b].at[1], wo, osem)

    def wait_out(g, b):
        we, wo = out_windows(g)
        pltpu.make_async_copy(rows_v.at[b].at[0], we, osem).wait()
        pltpu.make_async_copy(rows_v.at[b].at[1], wo, osem).wait()

    def build_body(g, carry):
        build_lists(g)
        return carry

    lax.fori_loop(0, NGROUPS, build_body, 0)

    for g in range(3):
        fire_gathers(g, g)

    def proc(g, first, last):
        b = lax.rem(g, 4) if not isinstance(g, int) else g % 4
        wait_gathers(g, b)
        start_out(g, b)
        if not first:
            wait_out(g - 1, lax.rem(g - 1, 4) if not isinstance(g, int) else (g - 1) % 4)
        if not last:
            fire_gathers(g + 3, lax.rem(g + 3, 4) if not isinstance(g, int) else (g + 3) % 4)

    proc(0, True, False)

    def body(g, carry):
        proc(g, False, False)
        return carry

    lax.fori_loop(1, NGROUPS - 3, body, 0)

    proc(NGROUPS - 3, False, True)
    proc(NGROUPS - 2, False, True)
    proc(NGROUPS - 1, False, True)
    wait_out(NGROUPS - 1, (NGROUPS - 1) % 4)


def kernel(input_, weight):
    idx = input_.reshape(BATCH // 128, 128)
    out = _gather_kernel(idx, weight)
    return out.reshape(B0, B1, EMBEDDING_DIM)
